# baseline (speedup 1.0000x reference)
import functools

import jax
import jax.numpy as jnp
from jax import lax
from jax.experimental import pallas as pl
from jax.experimental.pallas import tpu as pltpu
from jax.experimental.pallas import tpu_sc as plsc

RESCALE = 5.0
E_BLK = 1280
SPAN = 128
SPECIES_PAD = 128


def _species_gather_sc(node_species, idx_all):
    n_nodes = node_species.shape[0]
    n_idx = idx_all.shape[0]
    info = plsc.get_sparse_core_info()
    nw = info.num_cores * info.num_subcores
    lanes = info.num_lanes
    b_per_w = n_idx // nw
    assert n_idx % (nw * lanes) == 0 and b_per_w % 8 == 0
    assert n_nodes % lanes == 0 and lanes == 16
    node_species = node_species.reshape(n_nodes // lanes, lanes)
    mesh = plsc.VectorSubcoreMesh(core_axis_name="c", subcore_axis_name="s")

    @functools.partial(
        pl.kernel, mesh=mesh,
        compiler_params=pltpu.CompilerParams(needs_layout_passes=False),
        out_type=jax.ShapeDtypeStruct((n_idx,), jnp.float32),
        scratch_types=[
            pltpu.VMEM((n_nodes // lanes, lanes), jnp.int32),
            pltpu.VMEM((b_per_w,), jnp.int32),
            pltpu.VMEM((b_per_w,), jnp.float32),
        ],
    )
    def gather_k(species_hbm, idx_hbm, out_hbm, table_v, idx_v, out_v):
        wid = lax.axis_index("s") * info.num_cores + lax.axis_index("c")
        base = wid * b_per_w
        pltpu.sync_copy(species_hbm, table_v)
        pltpu.sync_copy(idx_hbm.at[pl.ds(base, b_per_w)], idx_v)

        def body(k, carry):
            idxv = idx_v[pl.ds(k * lanes, lanes)]
            vals = plsc.load_gather(
                table_v, [lax.shift_right_logical(idxv, 4),
                          lax.bitwise_and(idxv, 15)])
            out_v[pl.ds(k * lanes, lanes)] = vals.astype(jnp.float32)
            return carry

        lax.fori_loop(0, b_per_w // lanes, body, 0)
        pltpu.sync_copy(out_v, out_hbm.at[pl.ds(base, b_per_w)])

    return gather_k(node_species, idx_all)


def _ln_silu_t(x, g, b):
    m = jnp.mean(x, axis=0, keepdims=True)
    d = x - m
    v = jnp.mean(d * d, axis=0, keepdims=True)
    y = d * jax.lax.rsqrt(v + 1e-6) * g + b
    return y * jax.nn.sigmoid(y)


def _edge_kernel(meta_ref, ee_ref, wig_ref, sps_ref, spr_ref, rcv_ref,
                 T1s_ref, T1r_ref, W1a_ref, W2_ref, W3_ref,
                 b1_ref, g1_ref, be1_ref, b2_ref, g2_ref, be2_ref, b3_ref,
                 out_ref, *, n_nodes, num_coef, sc):
    i = pl.program_id(0)

    @pl.when(i == 0)
    def _():
        out_ref[...] = jnp.zeros_like(out_ref)

    iota_sp = lax.broadcasted_iota(jnp.int32, (SPECIES_PAD, 1), 0).astype(
        jnp.float32)
    oh_s = (sps_ref[0] == iota_sp).astype(jnp.float32)
    oh_r = (spr_ref[0] == iota_sp).astype(jnp.float32)

    x = (lax.dot_general(W1a_ref[...], ee_ref[...], (((1,), (1,)), ((), ())),
                         preferred_element_type=jnp.float32)
         + jnp.dot(T1s_ref[...], oh_s, preferred_element_type=jnp.float32)
         + jnp.dot(T1r_ref[...], oh_r, preferred_element_type=jnp.float32)
         + b1_ref[...])
    x = _ln_silu_t(x, g1_ref[...], be1_ref[...])
    x = jnp.dot(W2_ref[...], x, preferred_element_type=jnp.float32) + b2_ref[...]
    x = _ln_silu_t(x, g2_ref[...], be2_ref[...])
    y = jnp.dot(W3_ref[...], x, preferred_element_type=jnp.float32) + b3_ref[...]

    wig = jnp.transpose(wig_ref[...])
    chunks = []
    for ii in range(num_coef):
        acc = jnp.broadcast_to(wig[num_coef * ii + 0:num_coef * ii + 1, :],
                               (sc, E_BLK)) * y[0:sc, :]
        acc += jnp.broadcast_to(wig[num_coef * ii + 1:num_coef * ii + 2, :],
                                (sc, E_BLK)) * y[sc:2 * sc, :]
        acc += jnp.broadcast_to(wig[num_coef * ii + 2:num_coef * ii + 3, :],
                                (sc, E_BLK)) * y[2 * sc:3 * sc, :]
        chunks.append(acc)
    V = jnp.concatenate(chunks, axis=0)

    rcv = rcv_ref[...]
    r_lo = meta_ref[i, 0]
    nw = meta_ref[i, 1]
    iota_span = lax.broadcasted_iota(jnp.int32, (1, SPAN), 1).astype(
        jnp.float32)

    def body(w, carry):
        start = r_lo + w * SPAN
        s_w = jnp.minimum(start, n_nodes - SPAN)
        s_w = pl.multiple_of(s_w, 8)
        sf = s_w.astype(jnp.float32)
        startf = start.astype(jnp.float32)
        oh = ((rcv == sf + iota_span)
              & (rcv >= startf) & (rcv < startf + SPAN))
        p = jnp.dot(V, oh.astype(jnp.float32),
                    preferred_element_type=jnp.float32)
        out_ref[pl.ds(s_w, SPAN), :] += p.T
        return carry

    lax.fori_loop(0, nw, body, 0)


def kernel(node_species, edge_embeds, senders, receivers, wigner_inv,
           embed_table, W1, b1, g1, be1, W2, b2, g2, be2, W3, b3):
    n_nodes = node_species.shape[0]
    n_edges = edge_embeds.shape[0]
    d_edge = edge_embeds.shape[1]
    hid = W1.shape[1]
    out_d = W3.shape[1]
    sc = out_d // 3
    num_coef = wigner_inv.shape[1]
    n_species = embed_table.shape[0]
    nb = n_edges // E_BLK

    idx_all = jnp.concatenate([senders, receivers]).astype(jnp.int32)
    sp_all = _species_gather_sc(node_species.astype(jnp.int32), idx_all)
    sp_s = sp_all[:n_edges]
    sp_r = sp_all[n_edges:]

    table_s = embed_table[:, :hid]
    table_r = embed_table[:, hid:]
    W1aT = W1[:d_edge].T
    T1sT = jnp.zeros((hid, SPECIES_PAD), jnp.float32).at[:, :n_species].set(
        (table_s @ W1[d_edge:d_edge + hid]).T)
    T1rT = jnp.zeros((hid, SPECIES_PAD), jnp.float32).at[:, :n_species].set(
        (table_r @ W1[d_edge + hid:]).T)
    W2T = W2.T
    W3T = W3.T * (1.0 / RESCALE)
    b3c = (b3 * (1.0 / RESCALE)).reshape(out_d, 1)

    wig_flat = wigner_inv.reshape(n_edges, num_coef * num_coef)
    sp_s_row = sp_s.reshape(nb, 1, E_BLK)
    sp_r_row = sp_r.reshape(nb, 1, E_BLK)
    rcv_col = receivers.astype(jnp.float32).reshape(n_edges, 1)

    r2 = receivers.reshape(nb, E_BLK)
    r_lo_al = (r2[:, 0] // 8) * 8
    nw = (r2[:, -1] - r_lo_al) // SPAN + 1
    meta = jnp.stack([r_lo_al, nw], axis=1).astype(jnp.int32)

    small = lambda shape: pl.BlockSpec(shape, lambda i: (0,) * len(shape))
    out = pl.pallas_call(
        functools.partial(_edge_kernel, n_nodes=n_nodes, num_coef=num_coef,
                          sc=sc),
        grid=(nb,),
        in_specs=[
            pl.BlockSpec(memory_space=pltpu.SMEM),
            pl.BlockSpec((E_BLK, d_edge), lambda i: (i, 0)),
            pl.BlockSpec((E_BLK, num_coef * num_coef), lambda i: (i, 0)),
            pl.BlockSpec((1, 1, E_BLK), lambda i: (i, 0, 0)),
            pl.BlockSpec((1, 1, E_BLK), lambda i: (i, 0, 0)),
            pl.BlockSpec((E_BLK, 1), lambda i: (i, 0)),
            small((hid, SPECIES_PAD)), small((hid, SPECIES_PAD)),
            small((hid, d_edge)), small((hid, hid)), small((out_d, hid)),
            small((hid, 1)), small((hid, 1)), small((hid, 1)),
            small((hid, 1)), small((hid, 1)), small((hid, 1)),
            small((out_d, 1)),
        ],
        out_specs=pl.BlockSpec((n_nodes, num_coef * sc), lambda i: (0, 0)),
        out_shape=jax.ShapeDtypeStruct((n_nodes, num_coef * sc), jnp.float32),
    )(meta, edge_embeds, wig_flat, sp_s_row, sp_r_row, rcv_col,
      T1sT, T1rT, W1aT, W2T, W3T,
      b1.reshape(hid, 1), g1.reshape(hid, 1), be1.reshape(hid, 1),
      b2.reshape(hid, 1), g2.reshape(hid, 1), be2.reshape(hid, 1), b3c)

    return out.reshape(n_nodes, num_coef, sc)


# speedup vs baseline: 1.1526x; 1.1526x over previous
import functools

import jax
import jax.numpy as jnp
from jax import lax
from jax.experimental import pallas as pl
from jax.experimental.pallas import tpu as pltpu
from jax.experimental.pallas import tpu_sc as plsc

RESCALE = 5.0
E_BLK = 1280
SPAN = 128
SPECIES_PAD = 128


def _species_gather_sc(node_species, idx_all):
    n_nodes = node_species.shape[0]
    n_idx = idx_all.shape[0]
    info = plsc.get_sparse_core_info()
    nw = info.num_cores * info.num_subcores
    lanes = info.num_lanes
    b_per_w = n_idx // nw
    assert n_idx % (nw * lanes) == 0 and b_per_w % 8 == 0
    assert n_nodes % lanes == 0 and lanes == 16
    node_species = node_species.reshape(n_nodes // lanes, lanes)
    mesh = plsc.VectorSubcoreMesh(core_axis_name="c", subcore_axis_name="s")

    @functools.partial(
        pl.kernel, mesh=mesh,
        compiler_params=pltpu.CompilerParams(needs_layout_passes=False),
        out_type=jax.ShapeDtypeStruct((n_idx,), jnp.float32),
        scratch_types=[
            pltpu.VMEM((n_nodes // lanes, lanes), jnp.int32),
            pltpu.VMEM((b_per_w,), jnp.int32),
            pltpu.VMEM((b_per_w,), jnp.float32),
        ],
    )
    def gather_k(species_hbm, idx_hbm, out_hbm, table_v, idx_v, out_v):
        wid = lax.axis_index("s") * info.num_cores + lax.axis_index("c")
        base = wid * b_per_w
        pltpu.sync_copy(species_hbm, table_v)
        pltpu.sync_copy(idx_hbm.at[pl.ds(base, b_per_w)], idx_v)

        def body(k, carry):
            idxv = idx_v[pl.ds(k * lanes, lanes)]
            vals = plsc.load_gather(
                table_v, [lax.shift_right_logical(idxv, 4),
                          lax.bitwise_and(idxv, 15)])
            out_v[pl.ds(k * lanes, lanes)] = vals.astype(jnp.float32)
            return carry

        lax.fori_loop(0, b_per_w // lanes, body, 0)
        pltpu.sync_copy(out_v, out_hbm.at[pl.ds(base, b_per_w)])

    return gather_k(node_species, idx_all)


def _ln_silu_t(x, g, b):
    m = jnp.mean(x, axis=0, keepdims=True)
    d = x - m
    v = jnp.mean(d * d, axis=0, keepdims=True)
    y = d * jax.lax.rsqrt(v + 1e-6) * g + b
    return y * jax.nn.sigmoid(y)


def _edge_kernel(meta_ref, ee_ref, wig_ref, sps_ref, spr_ref, rcv_ref,
                 T1s_ref, T1r_ref, W1a_ref, W2_ref, W3_ref,
                 b1_ref, g1_ref, be1_ref, b2_ref, g2_ref, be2_ref, b3_ref,
                 out_ref, *, n_nodes, num_coef, sc):
    i = pl.program_id(0)

    @pl.when(i == 0)
    def _():
        out_ref[...] = jnp.zeros_like(out_ref)

    iota_sp = lax.broadcasted_iota(jnp.int32, (SPECIES_PAD, 1), 0).astype(
        jnp.float32)
    oh_s = (sps_ref[0] == iota_sp).astype(jnp.float32)
    oh_r = (spr_ref[0] == iota_sp).astype(jnp.float32)

    x = (lax.dot_general(W1a_ref[...], ee_ref[...], (((1,), (1,)), ((), ())),
                         preferred_element_type=jnp.float32)
         + jnp.dot(T1s_ref[...], oh_s, preferred_element_type=jnp.float32)
         + jnp.dot(T1r_ref[...], oh_r, preferred_element_type=jnp.float32)
         + b1_ref[...])
    x = _ln_silu_t(x, g1_ref[...], be1_ref[...])
    x = jnp.dot(W2_ref[...], x, preferred_element_type=jnp.float32) + b2_ref[...]
    x = _ln_silu_t(x, g2_ref[...], be2_ref[...])
    y = jnp.dot(W3_ref[...], x, preferred_element_type=jnp.float32) + b3_ref[...]

    wig = wig_ref[...]
    chunks = []
    for ii in range(num_coef):
        acc = jnp.broadcast_to(wig[3 * ii + 0:3 * ii + 1, :],
                               (sc, E_BLK)) * y[0:sc, :]
        acc += jnp.broadcast_to(wig[3 * ii + 1:3 * ii + 2, :],
                                (sc, E_BLK)) * y[sc:2 * sc, :]
        acc += jnp.broadcast_to(wig[3 * ii + 2:3 * ii + 3, :],
                                (sc, E_BLK)) * y[2 * sc:3 * sc, :]
        chunks.append(acc)
    V = jnp.concatenate(chunks, axis=0)

    rcv = rcv_ref[...]
    r_lo = meta_ref[i, 0]
    nw = meta_ref[i, 1]
    iota_span = lax.broadcasted_iota(jnp.int32, (1, SPAN), 1).astype(
        jnp.float32)

    def body(w, carry):
        start = r_lo + w * SPAN
        s_w = jnp.minimum(start, n_nodes - SPAN)
        s_w = pl.multiple_of(s_w, 8)
        sf = s_w.astype(jnp.float32)
        startf = start.astype(jnp.float32)
        oh = ((rcv == sf + iota_span)
              & (rcv >= startf) & (rcv < startf + SPAN))
        p = jnp.dot(V, oh.astype(jnp.float32),
                    preferred_element_type=jnp.float32)
        out_ref[pl.ds(s_w, SPAN), :] += p.T
        return carry

    lax.fori_loop(0, nw, body, 0)


def kernel(node_species, edge_embeds, senders, receivers, wigner_inv,
           embed_table, W1, b1, g1, be1, W2, b2, g2, be2, W3, b3):
    n_nodes = node_species.shape[0]
    n_edges = edge_embeds.shape[0]
    d_edge = edge_embeds.shape[1]
    hid = W1.shape[1]
    out_d = W3.shape[1]
    sc = out_d // 3
    num_coef = wigner_inv.shape[1]
    n_species = embed_table.shape[0]
    nb = n_edges // E_BLK

    idx_all = jnp.concatenate([senders, receivers]).astype(jnp.int32)
    sp_all = _species_gather_sc(node_species.astype(jnp.int32), idx_all)
    sp_s = sp_all[:n_edges]
    sp_r = sp_all[n_edges:]

    table_s = embed_table[:, :hid]
    table_r = embed_table[:, hid:]
    W1aT = W1[:d_edge].T
    T1sT = jnp.zeros((hid, SPECIES_PAD), jnp.float32).at[:, :n_species].set(
        (table_s @ W1[d_edge:d_edge + hid]).T)
    T1rT = jnp.zeros((hid, SPECIES_PAD), jnp.float32).at[:, :n_species].set(
        (table_r @ W1[d_edge + hid:]).T)
    W2T = W2.T
    W3T = W3.T * (1.0 / RESCALE)
    b3c = (b3 * (1.0 / RESCALE)).reshape(out_d, 1)

    wig_t = wigner_inv[:, :, :3].reshape(n_edges, 3 * num_coef).T
    sp_s_row = sp_s.reshape(nb, 1, E_BLK)
    sp_r_row = sp_r.reshape(nb, 1, E_BLK)
    rcv_col = receivers.astype(jnp.float32).reshape(n_edges, 1)

    r2 = receivers.reshape(nb, E_BLK)
    r_lo_al = (r2[:, 0] // 8) * 8
    nw = (r2[:, -1] - r_lo_al) // SPAN + 1
    meta = jnp.stack([r_lo_al, nw], axis=1).astype(jnp.int32)

    small = lambda shape: pl.BlockSpec(shape, lambda i: (0,) * len(shape))
    out = pl.pallas_call(
        functools.partial(_edge_kernel, n_nodes=n_nodes, num_coef=num_coef,
                          sc=sc),
        grid=(nb,),
        in_specs=[
            pl.BlockSpec(memory_space=pltpu.SMEM),
            pl.BlockSpec((E_BLK, d_edge), lambda i: (i, 0)),
            pl.BlockSpec((3 * num_coef, E_BLK), lambda i: (0, i)),
            pl.BlockSpec((1, 1, E_BLK), lambda i: (i, 0, 0)),
            pl.BlockSpec((1, 1, E_BLK), lambda i: (i, 0, 0)),
            pl.BlockSpec((E_BLK, 1), lambda i: (i, 0)),
            small((hid, SPECIES_PAD)), small((hid, SPECIES_PAD)),
            small((hid, d_edge)), small((hid, hid)), small((out_d, hid)),
            small((hid, 1)), small((hid, 1)), small((hid, 1)),
            small((hid, 1)), small((hid, 1)), small((hid, 1)),
            small((out_d, 1)),
        ],
        out_specs=pl.BlockSpec((n_nodes, num_coef * sc), lambda i: (0, 0)),
        out_shape=jax.ShapeDtypeStruct((n_nodes, num_coef * sc), jnp.float32),
    )(meta, edge_embeds, wig_t, sp_s_row, sp_r_row, rcv_col,
      T1sT, T1rT, W1aT, W2T, W3T,
      b1.reshape(hid, 1), g1.reshape(hid, 1), be1.reshape(hid, 1),
      b2.reshape(hid, 1), g2.reshape(hid, 1), be2.reshape(hid, 1), b3c)

    return out.reshape(n_nodes, num_coef, sc)
